# baseline (speedup 1.0000x reference)
import jax
import jax.numpy as jnp
from jax.experimental import pallas as pl
from jax.experimental.pallas import tpu as pltpu

NUM_EMB_ = 1024
DIM_ = 64
CC_ = 0.6
TOK_BLOCK = 1024


def _vq_block_kernel(x_ref, emb_ref, w_ref, e2_ref, cols_ref,
                     q_ref, idx_ref, ps_ref):
    x = x_ref[...]
    emb = emb_ref[...]
    w = w_ref[...]
    e2 = e2_ref[...]
    cols = cols_ref[...]
    dot = jax.lax.dot_general(x, emb, (((1,), (1,)), ((), ())),
                              preferred_element_type=jnp.float32)
    x2 = jnp.sum(x * x, axis=1, keepdims=True)
    dist = (x2 + e2 - 2.0 * dot) * w
    m = jnp.min(dist, axis=1, keepdims=True)
    u = (dist == m).astype(jnp.float32)
    s = jax.lax.dot_general(u, cols, (((1,), (0,)), ((), ())),
                            preferred_element_type=jnp.float32)
    cnt = s[:, 0]
    has_tie = jnp.max(cnt) > 1.5

    @pl.when(jnp.logical_not(has_tie))
    def _unique():
        idx = (8.0 * s[:, 1] + s[:, 2]).astype(jnp.int32)
        q = jax.lax.dot_general(u, emb, (((1,), (0,)), ((), ())),
                                preferred_element_type=jnp.float32)
        q_ref[...] = q
        idx_ref[0, 0, :] = idx
        d = q - x
        ps_ref[...] = jnp.full((1, 1, 128), jnp.sum(d * d), dtype=jnp.float32)

    @pl.when(has_tie)
    def _tied():
        kio = jax.lax.broadcasted_iota(jnp.int32, dist.shape, 1)
        idx = jnp.min(jnp.where(dist == m, kio, NUM_EMB_), axis=1)
        oh = (kio == idx[:, None]).astype(jnp.float32)
        q = jax.lax.dot_general(oh, emb, (((1,), (0,)), ((), ())),
                                preferred_element_type=jnp.float32)
        q_ref[...] = q
        idx_ref[0, 0, :] = idx
        d = q - x
        ps_ref[...] = jnp.full((1, 1, 128), jnp.sum(d * d), dtype=jnp.float32)


def kernel(inputs, emb_weight, scaling):
    B, S, D = inputs.shape
    K = emb_weight.shape[0]
    N = B * S
    G = N // TOK_BLOCK
    flat = inputs.reshape(N, D)
    hr_values = jnp.linspace(40.0, 180.0, K)
    w = (1.0 + scaling * ((hr_values - 100.0) / 70.0)).reshape(1, K)
    e2 = jnp.sum(emb_weight ** 2, axis=1).reshape(1, K)
    k = jnp.arange(K, dtype=jnp.float32)
    cols = jnp.zeros((K, 128), jnp.float32)
    cols = cols.at[:, 0].set(1.0).at[:, 1].set(jnp.floor(k / 8.0)).at[:, 2].set(k % 8.0)

    q, idx3, ps = pl.pallas_call(
        _vq_block_kernel,
        grid=(G,),
        in_specs=[
            pl.BlockSpec((TOK_BLOCK, D), lambda i: (i, 0)),
            pl.BlockSpec((K, D), lambda i: (0, 0)),
            pl.BlockSpec((1, K), lambda i: (0, 0)),
            pl.BlockSpec((1, K), lambda i: (0, 0)),
            pl.BlockSpec((K, 128), lambda i: (0, 0)),
        ],
        out_specs=[
            pl.BlockSpec((TOK_BLOCK, D), lambda i: (i, 0)),
            pl.BlockSpec((1, 1, TOK_BLOCK), lambda i: (i, 0, 0)),
            pl.BlockSpec((1, 1, 128), lambda i: (i, 0, 0)),
        ],
        out_shape=[
            jax.ShapeDtypeStruct((N, D), jnp.float32),
            jax.ShapeDtypeStruct((G, 1, TOK_BLOCK), jnp.int32),
            jax.ShapeDtypeStruct((G, 1, 128), jnp.float32),
        ],
        compiler_params=pltpu.CompilerParams(
            dimension_semantics=("arbitrary",),
        ),
    )(flat, emb_weight, w, e2, cols)

    loss = (1.0 + CC_) * jnp.sum(ps[:, 0, 0]) / (N * D)
    quantized_st = q.reshape(inputs.shape)
    encoding_indices = idx3.reshape(B, S)
    return (quantized_st, loss, encoding_indices)


# speedup vs baseline: 1.2242x; 1.2242x over previous
import jax
import jax.numpy as jnp
from jax.experimental import pallas as pl
from jax.experimental.pallas import tpu as pltpu

NUM_EMB_ = 1024
DIM_ = 64
CC_ = 0.6
TOK_BLOCK = 2048


def _vq_block_kernel(x_ref, emb_ref, w_ref, e2_ref, q_ref, idx_ref, ps_ref):
    x = x_ref[...]
    emb = emb_ref[...]
    w = w_ref[...]
    e2 = e2_ref[...]
    dot = jax.lax.dot_general(x, emb, (((1,), (1,)), ((), ())),
                              preferred_element_type=jnp.float32)
    x2 = jnp.sum(x * x, axis=1, keepdims=True)
    dist = (x2 + e2 - 2.0 * dot) * w
    m = jnp.min(dist, axis=1, keepdims=True)
    kio = jax.lax.broadcasted_iota(jnp.int32, dist.shape, 1)
    idx = jnp.min(jnp.where(dist == m, kio, NUM_EMB_), axis=1)
    oh = (kio == idx[:, None]).astype(jnp.float32)
    q = jax.lax.dot_general(oh, emb, (((1,), (0,)), ((), ())),
                            preferred_element_type=jnp.float32)
    q_ref[...] = q
    idx_ref[0, 0, :] = idx
    d = q - x
    ps_ref[...] = jnp.full((1, 1, 128), jnp.sum(d * d), dtype=jnp.float32)


def kernel(inputs, emb_weight, scaling):
    B, S, D = inputs.shape
    K = emb_weight.shape[0]
    N = B * S
    G = N // TOK_BLOCK
    flat = inputs.reshape(N, D)
    hr_values = jnp.linspace(40.0, 180.0, K)
    w = (1.0 + scaling * ((hr_values - 100.0) / 70.0)).reshape(1, K)
    e2 = jnp.sum(emb_weight ** 2, axis=1).reshape(1, K)

    q, idx3, ps = pl.pallas_call(
        _vq_block_kernel,
        grid=(G,),
        in_specs=[
            pl.BlockSpec((TOK_BLOCK, D), lambda i: (i, 0)),
            pl.BlockSpec((K, D), lambda i: (0, 0)),
            pl.BlockSpec((1, K), lambda i: (0, 0)),
            pl.BlockSpec((1, K), lambda i: (0, 0)),
        ],
        out_specs=[
            pl.BlockSpec((TOK_BLOCK, D), lambda i: (i, 0)),
            pl.BlockSpec((1, 1, TOK_BLOCK), lambda i: (i, 0, 0)),
            pl.BlockSpec((1, 1, 128), lambda i: (i, 0, 0)),
        ],
        out_shape=[
            jax.ShapeDtypeStruct((N, D), jnp.float32),
            jax.ShapeDtypeStruct((G, 1, TOK_BLOCK), jnp.int32),
            jax.ShapeDtypeStruct((G, 1, 128), jnp.float32),
        ],
        compiler_params=pltpu.CompilerParams(
            dimension_semantics=("arbitrary",),
        ),
    )(flat, emb_weight, w, e2)

    loss = (1.0 + CC_) * jnp.sum(ps[:, 0, 0]) / (N * D)
    quantized_st = q.reshape(inputs.shape)
    encoding_indices = idx3.reshape(B, S)
    return (quantized_st, loss, encoding_indices)
